# baseline (speedup 1.0000x reference)
import functools

import jax
import jax.numpy as jnp
from jax import lax
from jax.experimental import pallas as pl
from jax.experimental.pallas import tpu as pltpu

AUG_P = 0.5
DROP_P = 0.15
MIN_FEATURES = 1


def _mask_kernel(s_ref, m_ref, aug_ref, keep_ref, *, F, B):
    m = m_ref[...].T > 0
    bits = lax.bitcast_convert_type(s_ref[...].T, jnp.int32)
    bits = jnp.where(m, bits, jnp.int32(0x7FFFFFFF))

    n_avail = jnp.sum(m.astype(jnp.int32), axis=0, keepdims=True)
    k = (n_avail.astype(jnp.float32) * DROP_P).astype(jnp.int32)
    k = jnp.minimum(k, n_avail - MIN_FEATURES)
    aug = aug_ref[...].T < AUG_P
    k = jnp.where((n_avail > MIN_FEATURES) & aug & (k > 0), k, 0)

    ans = jnp.zeros((1, B), jnp.int32)
    for bit in range(30, -1, -1):
        test = ans + jnp.int32(1 << bit)
        cnt = jnp.sum((bits < test).astype(jnp.int32), axis=0, keepdims=True)
        ans = jnp.where(cnt < k, test, ans)

    c_lt = jnp.sum((bits < ans).astype(jnp.int32), axis=0, keepdims=True)
    eq = bits == ans
    fi = lax.broadcasted_iota(jnp.int32, (F, F), 0)
    fj = lax.broadcasted_iota(jnp.int32, (F, F), 1)
    tril = (fj < fi).astype(jnp.float32)
    eq_before = jax.lax.dot(
        tril, eq.astype(jnp.float32), precision=jax.lax.Precision.HIGHEST
    ).astype(jnp.int32)
    drop = m & ((bits < ans) | (eq & ((c_lt + eq_before) < k)))
    keep_ref[...] = (1.0 - drop.astype(jnp.float32)).T


def _copy_kernel(x_ref, k_ref, o_ref):
    o_ref[...] = x_ref[...] * k_ref[...][:, :, None]


def kernel(input_features, attention_mask):
    B, F, C = input_features.shape
    key = jax.random.key(42)
    k1, k2 = jax.random.split(key)
    aug_u = jax.random.uniform(k1, (B,)).reshape(B, 1)
    scores = jax.random.uniform(k2, (B, F))
    mask_i32 = attention_mask.astype(jnp.int32)

    keep = pl.pallas_call(
        functools.partial(_mask_kernel, F=F, B=B),
        out_shape=jax.ShapeDtypeStruct((B, F), jnp.float32),
    )(scores, mask_i32, aug_u)

    BBLK = 32
    grid = (B // BBLK,)
    out = pl.pallas_call(
        _copy_kernel,
        grid=grid,
        compiler_params=pltpu.CompilerParams(
            dimension_semantics=("parallel",),
        ),
        in_specs=[
            pl.BlockSpec((BBLK, F, C), lambda i: (i, 0, 0)),
            pl.BlockSpec((BBLK, F), lambda i: (i, 0)),
        ],
        out_specs=pl.BlockSpec((BBLK, F, C), lambda i: (i, 0, 0)),
        out_shape=jax.ShapeDtypeStruct((B, F, C), input_features.dtype),
    )(input_features, keep)
    return out


# speedup vs baseline: 1.0711x; 1.0711x over previous
import functools

import jax
import jax.numpy as jnp
from jax import lax
from jax.experimental import pallas as pl
from jax.experimental.pallas import tpu as pltpu

AUG_P = 0.5
DROP_P = 0.15
MIN_FEATURES = 1


def _mask_kernel(s_ref, m_ref, aug_ref, keep_ref, *, F, B):
    m = m_ref[...].T > 0
    bits = lax.bitcast_convert_type(s_ref[...].T, jnp.int32)
    bits = jnp.where(m, bits, jnp.int32(0x7FFFFFFF))

    n_avail = jnp.sum(m.astype(jnp.int32), axis=0, keepdims=True)
    k = (n_avail.astype(jnp.float32) * DROP_P).astype(jnp.int32)
    k = jnp.minimum(k, n_avail - MIN_FEATURES)
    aug = aug_ref[...].T < AUG_P
    k = jnp.where((n_avail > MIN_FEATURES) & aug & (k > 0), k, 0)

    ans = jnp.zeros((1, B), jnp.int32)
    for bit in range(30, -1, -1):
        test = ans + jnp.int32(1 << bit)
        cnt = jnp.sum((bits < test).astype(jnp.int32), axis=0, keepdims=True)
        ans = jnp.where(cnt < k, test, ans)

    c_lt = jnp.sum((bits < ans).astype(jnp.int32), axis=0, keepdims=True)
    eq = bits == ans
    fi = lax.broadcasted_iota(jnp.int32, (F, F), 0)
    fj = lax.broadcasted_iota(jnp.int32, (F, F), 1)
    tril = (fj < fi).astype(jnp.float32)
    eq_before = jax.lax.dot(
        tril, eq.astype(jnp.float32), precision=jax.lax.Precision.HIGHEST
    ).astype(jnp.int32)
    drop = m & ((bits < ans) | (eq & ((c_lt + eq_before) < k)))
    keep_ref[...] = (1.0 - drop.astype(jnp.float32)).T


def _copy_kernel(x_ref, k_ref, o_ref, *, BBLK):
    i = pl.program_id(0)
    kb = k_ref[pl.ds(i * BBLK, BBLK), :]
    o_ref[...] = x_ref[...] * kb[:, :, None]


def kernel(input_features, attention_mask):
    B, F, C = input_features.shape
    key = jax.random.key(42)
    k1, k2 = jax.random.split(key)
    aug_u = jax.random.uniform(k1, (B,)).reshape(B, 1)
    scores = jax.random.uniform(k2, (B, F))
    mask_i32 = attention_mask.astype(jnp.int32)

    keep = pl.pallas_call(
        functools.partial(_mask_kernel, F=F, B=B),
        out_shape=jax.ShapeDtypeStruct((B, F), jnp.float32),
    )(scores, mask_i32, aug_u)

    BBLK = 128
    grid = (B // BBLK,)
    out = pl.pallas_call(
        functools.partial(_copy_kernel, BBLK=BBLK),
        grid=grid,
        compiler_params=pltpu.CompilerParams(
            dimension_semantics=("parallel",),
        ),
        in_specs=[
            pl.BlockSpec((BBLK, F, C), lambda i: (i, 0, 0)),
            pl.BlockSpec((B, F), lambda i: (0, 0)),
        ],
        out_specs=pl.BlockSpec((BBLK, F, C), lambda i: (i, 0, 0)),
        out_shape=jax.ShapeDtypeStruct((B, F, C), input_features.dtype),
    )(input_features, keep)
    return out


# speedup vs baseline: 1.0939x; 1.0214x over previous
import functools

import jax
import jax.numpy as jnp
from jax import lax
from jax.experimental import pallas as pl
from jax.experimental.pallas import tpu as pltpu

AUG_P = 0.5
DROP_P = 0.15
MIN_FEATURES = 1


def _mask_kernel(s_ref, m_ref, aug_ref, keep_ref, *, F, B):
    m = m_ref[...].T > 0
    bits = lax.bitcast_convert_type(s_ref[...].T, jnp.int32)
    bits = jnp.where(m, bits, jnp.int32(0x7FFFFFFF))

    n_avail = jnp.sum(m.astype(jnp.int32), axis=0, keepdims=True)
    k = (n_avail.astype(jnp.float32) * DROP_P).astype(jnp.int32)
    k = jnp.minimum(k, n_avail - MIN_FEATURES)
    aug = aug_ref[...].T < AUG_P
    k = jnp.where((n_avail > MIN_FEATURES) & aug & (k > 0), k, 0)

    ans = jnp.zeros((1, B), jnp.int32)
    for bit in range(30, -1, -1):
        test = ans + jnp.int32(1 << bit)
        cnt = jnp.sum((bits < test).astype(jnp.int32), axis=0, keepdims=True)
        ans = jnp.where(cnt < k, test, ans)

    c_lt = jnp.sum((bits < ans).astype(jnp.int32), axis=0, keepdims=True)
    eq = bits == ans
    fi = lax.broadcasted_iota(jnp.int32, (F, F), 0)
    fj = lax.broadcasted_iota(jnp.int32, (F, F), 1)
    tril = (fj < fi).astype(jnp.float32)
    eq_before = jax.lax.dot(
        tril, eq.astype(jnp.float32), precision=jax.lax.Precision.HIGHEST
    ).astype(jnp.int32)
    drop = m & ((bits < ans) | (eq & ((c_lt + eq_before) < k)))
    keep_ref[...] = (1.0 - drop.astype(jnp.float32)).T


NRING = 4


def _copy_manual(x_hbm, keep_hbm, o_hbm, kv,
                 i0, i1, i2, i3, o0, o1, o2, o3,
                 ks, si0, si1, si2, si3, so0, so1, so2, so3,
                 *, B, F, C, BBLK):
    NBLK = B // BBLK
    NROUND = NBLK // NRING
    ibufs = (i0, i1, i2, i3)
    obufs = (o0, o1, o2, o3)
    sin = (si0, si1, si2, si3)
    sout = (so0, so1, so2, so3)

    pltpu.make_async_copy(keep_hbm, kv, ks).start()

    def in_desc(p, j):
        return pltpu.make_async_copy(
            x_hbm.at[pl.ds(p * BBLK, BBLK)], ibufs[j], sin[j]
        )

    def out_desc(p, j):
        return pltpu.make_async_copy(
            obufs[j], o_hbm.at[pl.ds(p * BBLK, BBLK)], sout[j]
        )

    for q in range(NRING - 1):
        in_desc(q, q).start()
    pltpu.make_async_copy(keep_hbm, kv, ks).wait()

    def round_body(t, carry):
        for j in range(NRING):
            p = t * NRING + j

            j2 = (j + NRING - 1) % NRING

            @pl.when(p + NRING - 1 < NBLK)
            def _():
                in_desc(p + NRING - 1, j2).start()

            in_desc(p, j).wait()

            @pl.when(t > 0)
            def _():
                out_desc(p - NRING, j).wait()

            kb = kv[pl.ds(p * BBLK, BBLK), :]
            obufs[j][...] = ibufs[j][...] * kb[:, :, None]
            out_desc(p, j).start()
        return carry

    lax.fori_loop(0, NROUND, round_body, 0)
    for j in range(NRING):
        out_desc(NBLK - NRING + j, j).wait()


def kernel(input_features, attention_mask):
    B, F, C = input_features.shape
    key = jax.random.key(42)
    k1, k2 = jax.random.split(key)
    aug_u = jax.random.uniform(k1, (B,)).reshape(B, 1)
    scores = jax.random.uniform(k2, (B, F))
    mask_i32 = attention_mask.astype(jnp.int32)

    keep = pl.pallas_call(
        functools.partial(_mask_kernel, F=F, B=B),
        out_shape=jax.ShapeDtypeStruct((B, F), jnp.float32),
    )(scores, mask_i32, aug_u)

    BBLK = 32
    out = pl.pallas_call(
        functools.partial(_copy_manual, B=B, F=F, C=C, BBLK=BBLK),
        in_specs=[
            pl.BlockSpec(memory_space=pl.ANY),
            pl.BlockSpec(memory_space=pl.ANY),
        ],
        out_specs=pl.BlockSpec(memory_space=pl.ANY),
        out_shape=jax.ShapeDtypeStruct((B, F, C), input_features.dtype),
        scratch_shapes=(
            [pltpu.VMEM((B, F), jnp.float32)]
            + [pltpu.VMEM((BBLK, F, C), jnp.float32) for _ in range(8)]
            + [pltpu.SemaphoreType.DMA for _ in range(9)]
        ),
    )(input_features, keep)
    return out
